# baseline (speedup 1.0000x reference)
import functools

import jax
import jax.numpy as jnp
from jax import lax
from jax.experimental import pallas as pl
from jax.experimental.pallas import tpu as pltpu
from jax.experimental.pallas import tpu_sc as plsc

NC = 2
NS = 16
NW = NC * NS

C = 200
GROUPS = 13


def _select_chunk(scol_v, rows_v, out_v):
    iota = lax.iota(jnp.int32, 16)
    zeros = jnp.zeros((16,), jnp.int32)
    tailmask = iota < 8
    for g in range(GROUPS):
        rid = iota + g * 16
        sc = scol_v[pl.ds(g * 16, 16)]
        mask = None if g < GROUPS - 1 else tailmask
        for j in range(32):
            vals = plsc.load_gather(rows_v, [rid, sc + j], mask=mask)
            plsc.store_scatter(out_v, [zeros, rid, zeros + j], vals,
                               mask=mask)


def _body(n_chunks, q_hbm, scol_hbm, table_hbm, out_hbm,
          q_v0, q_v1, scol_v0, scol_v1, rows_v0, rows_v1, out_v0, out_v1,
          sem_q0, sem_q1, sem_c0, sem_c1, sem_g0, sem_g1, sem_s0, sem_s1):
    wid = lax.axis_index("s") * NC + lax.axis_index("c")
    base = wid * (n_chunks * C)
    row0 = wid * n_chunks
    q_v = (q_v0, q_v1)
    scol_v = (scol_v0, scol_v1)
    rows_v = (rows_v0, rows_v1)
    out_v = (out_v0, out_v1)
    sem_q = (sem_q0, sem_q1)
    sem_c = (sem_c0, sem_c1)
    sem_g = (sem_g0, sem_g1)
    sem_s = (sem_s0, sem_s1)

    def stage_start(c, s):
        pltpu.async_copy(q_hbm.at[pl.ds(base + c * C, C)], q_v[s], sem_q[s])
        pltpu.async_copy(scol_hbm.at[pl.ds(base + c * C, C)],
                         scol_v[s].at[pl.ds(0, C)], sem_c[s])

    def stage_wait(c, s):
        pltpu.make_async_copy(q_hbm.at[pl.ds(base + c * C, C)], q_v[s],
                              sem_q[s]).wait()
        pltpu.make_async_copy(scol_hbm.at[pl.ds(base + c * C, C)],
                              scol_v[s].at[pl.ds(0, C)], sem_c[s]).wait()

    def gather_start(s):
        pltpu.async_copy(table_hbm.at[q_v[s]], rows_v[s], sem_g[s])

    def gather_wait(s):
        pltpu.make_async_copy(table_hbm.at[q_v[s]], rows_v[s],
                              sem_g[s]).wait()

    def store_start(c, s):
        pltpu.async_copy(out_v[s], out_hbm.at[pl.ds(row0 + c, 1)], sem_s[s])

    def store_wait(c, s):
        pltpu.make_async_copy(out_v[s], out_hbm.at[pl.ds(row0 + c, 1)],
                              sem_s[s]).wait()

    stage_start(0, 0)
    stage_start(1, 1)

    def body(k, carry):
        a = 2 * k
        b = a + 1
        stage_wait(a, 0)
        gather_start(0)
        stage_wait(b, 1)
        gather_start(1)

        gather_wait(0)

        @pl.when(k > 0)
        def _():
            store_wait(a - 2, 0)

        _select_chunk(scol_v[0], rows_v[0], out_v[0])
        store_start(a, 0)

        @pl.when(k < (n_chunks // 2) - 1)
        def _():
            stage_start(a + 2, 0)

        gather_wait(1)

        @pl.when(k > 0)
        def _():
            store_wait(b - 2, 1)

        _select_chunk(scol_v[1], rows_v[1], out_v[1])
        store_start(b, 1)

        @pl.when(k < (n_chunks // 2) - 1)
        def _():
            stage_start(b + 2, 1)

        return carry

    lax.fori_loop(0, n_chunks // 2, body, 0)

    store_wait(n_chunks - 2, 0)
    store_wait(n_chunks - 1, 1)


def kernel(input, weight):
    B0, B1 = input.shape
    V, D = weight.shape
    N = B0 * B1
    n_chunks = N // (NW * C)
    assert n_chunks * NW * C == N and n_chunks % 2 == 0

    idx = input.reshape(N).astype(jnp.int32)
    q = idx >> 2
    scol = (idx & 3) * D
    wview = weight.reshape(V // 4, 4 * D)

    mesh = plsc.VectorSubcoreMesh(core_axis_name="c", subcore_axis_name="s")
    run = pl.kernel(
        functools.partial(_body, n_chunks),
        out_type=jax.ShapeDtypeStruct((B0, B1, D), jnp.float32),
        mesh=mesh,
        scratch_types=[
            pltpu.VMEM((C,), jnp.int32),
            pltpu.VMEM((C,), jnp.int32),
            pltpu.VMEM((GROUPS * 16,), jnp.int32),
            pltpu.VMEM((GROUPS * 16,), jnp.int32),
            pltpu.VMEM((C, 4 * D), jnp.float32),
            pltpu.VMEM((C, 4 * D), jnp.float32),
            pltpu.VMEM((1, B1, D), jnp.float32),
            pltpu.VMEM((1, B1, D), jnp.float32),
            pltpu.SemaphoreType.DMA,
            pltpu.SemaphoreType.DMA,
            pltpu.SemaphoreType.DMA,
            pltpu.SemaphoreType.DMA,
            pltpu.SemaphoreType.DMA,
            pltpu.SemaphoreType.DMA,
            pltpu.SemaphoreType.DMA,
            pltpu.SemaphoreType.DMA,
        ],
        compiler_params=pltpu.CompilerParams(needs_layout_passes=False),
    )
    return run(q, scol, wview)


# speedup vs baseline: 1.8769x; 1.8769x over previous
import functools

import jax
import jax.numpy as jnp
from jax import lax
from jax.experimental import pallas as pl
from jax.experimental.pallas import tpu as pltpu
from jax.experimental.pallas import tpu_sc as plsc

NC = 2
NS = 16
NW = NC * NS

CHUNK = 1600
NBUF = 2


def _gather_body(n_per_w, n_chunks, idx_hbm, table_hbm, out_hbm,
                 idx_v, rows_v, sem_i0, sem_i1, sem_g, sem_s0, sem_s1):
    wid = lax.axis_index("s") * NC + lax.axis_index("c")
    base = wid * n_per_w
    sem_i = (sem_i0, sem_i1)
    sem_s = (sem_s0, sem_s1)

    def start_idx(j, p):
        return pltpu.async_copy(
            idx_hbm.at[pl.ds(base + j * CHUNK, CHUNK)], idx_v.at[p], sem_i[p])

    idx_pending = {0: start_idx(0, 0), 1: start_idx(1, 1)}
    store_pending = {}

    for j in range(n_chunks):
        p = j % NBUF
        if j >= NBUF:
            store_pending.pop(j - NBUF).wait()
        idx_pending.pop(j).wait()
        gather = pltpu.async_copy(table_hbm.at[idx_v.at[p]], rows_v.at[p],
                                  sem_g)
        gather.wait()
        if j + NBUF < n_chunks:
            idx_pending[j + NBUF] = start_idx(j + NBUF, p)
        store_pending[j] = pltpu.async_copy(
            rows_v.at[p], out_hbm.at[pl.ds(base + j * CHUNK, CHUNK)], sem_s[p])

    for j in sorted(store_pending):
        store_pending.pop(j).wait()


def kernel(input, weight):
    B0, B1 = input.shape
    V, D = weight.shape
    N = B0 * B1
    assert N % (NW * CHUNK) == 0
    n_per_w = N // NW
    n_chunks = n_per_w // CHUNK

    idx = input.reshape(N).astype(jnp.int32)

    mesh = plsc.VectorSubcoreMesh(core_axis_name="c", subcore_axis_name="s")
    run = pl.kernel(
        functools.partial(_gather_body, n_per_w, n_chunks),
        out_type=jax.ShapeDtypeStruct((N, D), jnp.float32),
        mesh=mesh,
        scratch_types=[
            pltpu.VMEM((NBUF, CHUNK), jnp.int32),
            pltpu.VMEM((NBUF, CHUNK, D), jnp.float32),
            pltpu.SemaphoreType.DMA,
            pltpu.SemaphoreType.DMA,
            pltpu.SemaphoreType.DMA,
            pltpu.SemaphoreType.DMA,
            pltpu.SemaphoreType.DMA,
        ],
        compiler_params=pltpu.CompilerParams(use_tc_tiling_on_sc=False),
    )
    out = run(idx, weight)
    return out.reshape(B0, B1, D)
